# baseline (speedup 1.0000x reference)
import functools

import jax
import jax.numpy as jnp
from jax import lax
from jax.experimental import pallas as pl
from jax.experimental.pallas import tpu as pltpu

_U = 8
_CN = 16384


def _emb_ln_kernel(sp_ref,
                   tok_hbm,
                   pfx_ref,
                   pos_ref,
                   idv_ref,
                   segf_ref,
                   dseg_ref,
                   gamma_ref,
                   beta_ref,
                   o_ref,
                   gbuf,
                   ptile,
                   sems):
    S, D = pos_ref.shape
    core = pl.program_id(0)
    i = pl.program_id(1)
    nb = pl.num_programs(1)
    ntok = pl.num_programs(0) * nb * S
    slot = lax.rem(i, 2)
    b = core * nb + i

    def issue_tile(batch, sl):
        base = batch * S

        def chunk(k):
            r0 = k * _U
            for u in range(_U):
                r = r0 + u
                idx = sp_ref[base + r]

                @pl.when(idx >= _CN)
                def _():
                    pltpu.make_async_copy(
                        tok_hbm.at[pl.ds(idx, 1)],
                        gbuf.at[sl, pl.ds(r, 1)],
                        sems.at[sl]).start(priority=u % 2)

        pl.loop(0, S // _U)(chunk)

    @pl.when(i == 0)
    def _prime():
        issue_tile(b, 0)

    @pl.when(i + 1 < nb)
    def _prefetch():
        issue_tile(b + 1, 1 - slot)

    base = b * S

    def pchunk(k):
        r0 = k * _U
        for u in range(_U):
            r = r0 + u
            idx = sp_ref[base + r]
            ptile[r, 0] = pfx_ref[lax.min(idx, _CN - 1), 0]

    pl.loop(0, S // _U)(pchunk)

    n = sp_ref[ntok + b]

    @pl.when(n > 0)
    def _wait():
        pltpu.make_async_copy(ptile.at[pl.ds(0, n)],
                              ptile.at[pl.ds(0, n)],
                              sems.at[slot]).wait()

    from_vmem = idv_ref[0] < _CN
    tok = jnp.where(from_vmem, ptile[:, 0, :], gbuf[slot])
    emb = tok + pos_ref[...] + segf_ref[0] * dseg_ref[...]
    mean = jnp.mean(emb, axis=-1, keepdims=True)
    cen = emb - mean
    var = jnp.mean(cen * cen, axis=-1, keepdims=True)
    normed = cen * lax.rsqrt(var + 1e-5)
    o_ref[0] = normed * gamma_ref[...] + beta_ref[...]


@functools.partial(jax.jit, static_argnames=())
def kernel(x, seg, tok_tab, pos_tab, seg_tab, gamma, beta):
    B, S = x.shape
    V, D = tok_tab.shape
    assert B % 2 == 0
    nb = B // 2

    ids = jnp.clip(x.astype(jnp.int32), 0, V - 1)
    counts = jnp.sum((ids >= _CN).astype(jnp.int32), axis=1)
    sp = jnp.concatenate([ids.reshape(B * S), counts])
    tok3 = tok_tab.reshape(V, 1, D)
    pos2 = pos_tab[:S] + seg_tab[0][None, :]
    dseg = (seg_tab[1] - seg_tab[0]).reshape(1, D)
    idv = ids.reshape(B, S, 1)
    segf = seg.reshape(B, S, 1).astype(jnp.float32)
    gamma2 = gamma.reshape(1, D)
    beta2 = beta.reshape(1, D)

    grid_spec = pltpu.PrefetchScalarGridSpec(
        num_scalar_prefetch=1,
        grid=(2, nb),
        in_specs=[
            pl.BlockSpec(memory_space=pl.ANY),
            pl.BlockSpec((_CN, 1, D), lambda c, i, ids: (0, 0, 0)),
            pl.BlockSpec((S, D), lambda c, i, ids: (0, 0)),
            pl.BlockSpec((1, S, 1), lambda c, i, ids: (c * (B // 2) + i, 0, 0)),
            pl.BlockSpec((1, S, 1), lambda c, i, ids: (c * (B // 2) + i, 0, 0)),
            pl.BlockSpec((1, D), lambda c, i, ids: (0, 0)),
            pl.BlockSpec((1, D), lambda c, i, ids: (0, 0)),
            pl.BlockSpec((1, D), lambda c, i, ids: (0, 0)),
        ],
        out_specs=pl.BlockSpec((1, S, D), lambda c, i, ids: (c * (B // 2) + i, 0, 0)),
        scratch_shapes=[
            pltpu.VMEM((2, S, D), tok_tab.dtype),
            pltpu.VMEM((S, 1, D), tok_tab.dtype),
            pltpu.SemaphoreType.DMA((2,)),
        ],
    )

    return pl.pallas_call(
        _emb_ln_kernel,
        out_shape=jax.ShapeDtypeStruct((B, S, D), jnp.float32),
        grid_spec=grid_spec,
        compiler_params=pltpu.CompilerParams(
            dimension_semantics=("parallel", "arbitrary"),
            disable_bounds_checks=True,
            vmem_limit_bytes=64 * 1024 * 1024,
        ),
    )(sp, tok_tab, tok3, pos2, idv, segf, dseg, gamma2, beta2)


# speedup vs baseline: 2.7438x; 2.7438x over previous
import functools

import jax
import jax.numpy as jnp
from jax import lax
from jax.experimental import pallas as pl
from jax.experimental.pallas import tpu as pltpu

_U = 8


def _emb_ln_kernel(ids_ref,
                   tok_hbm,
                   pos_ref,
                   segf_ref,
                   dseg_ref,
                   gamma_ref,
                   beta_ref,
                   o_ref,
                   gbuf,
                   sems):
    S, D = pos_ref.shape
    core = pl.program_id(0)
    i = pl.program_id(1)
    nb = pl.num_programs(1)
    slot = lax.rem(i, 2)

    def issue_tile(batch, sl):
        base = batch * S

        def chunk(k):
            r0 = k * _U
            for u in range(_U):
                r = r0 + u
                idx = ids_ref[base + r]
                pltpu.make_async_copy(
                    tok_hbm.at[pl.ds(idx, 1)],
                    gbuf.at[sl, pl.ds(r, 1)],
                    sems.at[sl, u % 2]).start(priority=u % 2)

        pl.loop(0, S // _U)(chunk)

    b = core * nb + i

    @pl.when(i == 0)
    def _prime():
        issue_tile(b, 0)

    @pl.when(i + 1 < nb)
    def _prefetch():
        issue_tile(b + 1, 1 - slot)

    for p in range(2):
        pltpu.make_async_copy(tok_hbm.at[pl.ds(0, S // 2)],
                              gbuf.at[slot, pl.ds(0, S // 2)],
                              sems.at[slot, p]).wait()

    emb = gbuf[slot] + pos_ref[...] + segf_ref[0] * dseg_ref[...]
    mean = jnp.mean(emb, axis=-1, keepdims=True)
    cen = emb - mean
    var = jnp.mean(cen * cen, axis=-1, keepdims=True)
    normed = cen * lax.rsqrt(var + 1e-5)
    o_ref[0] = normed * gamma_ref[...] + beta_ref[...]


@functools.partial(jax.jit, static_argnames=())
def kernel(x, seg, tok_tab, pos_tab, seg_tab, gamma, beta):
    B, S = x.shape
    V, D = tok_tab.shape
    assert B % 2 == 0
    nb = B // 2

    ids_flat = jnp.clip(x.reshape(B * S).astype(jnp.int32), 0, V - 1)
    pos2 = pos_tab[:S] + seg_tab[0][None, :]
    dseg = (seg_tab[1] - seg_tab[0]).reshape(1, D)
    segf = seg.reshape(B, S, 1).astype(jnp.float32)
    gamma2 = gamma.reshape(1, D)
    beta2 = beta.reshape(1, D)

    grid_spec = pltpu.PrefetchScalarGridSpec(
        num_scalar_prefetch=1,
        grid=(2, nb),
        in_specs=[
            pl.BlockSpec(memory_space=pl.ANY),
            pl.BlockSpec((S, D), lambda c, i, ids: (0, 0)),
            pl.BlockSpec((1, S, 1), lambda c, i, ids: (c * (B // 2) + i, 0, 0)),
            pl.BlockSpec((1, D), lambda c, i, ids: (0, 0)),
            pl.BlockSpec((1, D), lambda c, i, ids: (0, 0)),
            pl.BlockSpec((1, D), lambda c, i, ids: (0, 0)),
        ],
        out_specs=pl.BlockSpec((1, S, D), lambda c, i, ids: (c * (B // 2) + i, 0, 0)),
        scratch_shapes=[
            pltpu.VMEM((2, S, D), tok_tab.dtype),
            pltpu.SemaphoreType.DMA((2, 2)),
        ],
    )

    return pl.pallas_call(
        _emb_ln_kernel,
        out_shape=jax.ShapeDtypeStruct((B, S, D), jnp.float32),
        grid_spec=grid_spec,
        compiler_params=pltpu.CompilerParams(
            dimension_semantics=("parallel", "arbitrary"),
            disable_bounds_checks=True,
        ),
    )(ids_flat, tok_tab, pos2, segf, dseg, gamma2, beta2)
